# baseline (speedup 1.0000x reference)
import functools
import math

import jax
import jax.numpy as jnp
from jax import lax
from jax.experimental import pallas as pl
from jax.experimental.pallas import tpu as pltpu
from jax.experimental.pallas import tpu_sc as plsc

D_EMB = 128
SCALE = math.sqrt(float(D_EMB))

_info = plsc.get_sparse_core_info()
_NC = _info.num_cores
_NS = _info.num_subcores
_NW = _NC * _NS

_CH = 128
_NBUF = 4
_NGRP = D_EMB // 32


def _pack_table(table):
    v = table.shape[0]
    tb = table.astype(jnp.bfloat16).reshape(v, _NGRP, 2, 16)
    st = jnp.stack([tb[:, :, 0, :], tb[:, :, 1, :]], axis=-1)
    return lax.bitcast_convert_type(st, jnp.uint32).reshape(v, D_EMB // 2)


@functools.partial(jax.jit, static_argnames=("nch",))
def _gather(idx, packed, nch):
    b_total = _NW * nch * _CH
    mesh = plsc.VectorSubcoreMesh(core_axis_name="c", subcore_axis_name="s")

    @functools.partial(
        pl.kernel,
        mesh=mesh,
        out_type=jax.ShapeDtypeStruct((b_total, D_EMB), jnp.float32),
        scratch_types=[
            pltpu.VMEM((nch, _CH), jnp.int32),
            pltpu.VMEM((_NBUF, _CH, D_EMB // 2), jnp.uint32),
            pltpu.VMEM((_NBUF, _CH, D_EMB), jnp.float32),
            pltpu.SemaphoreType.DMA((_NBUF,)),
            pltpu.SemaphoreType.DMA((_NBUF,)),
        ],
        compiler_params=pltpu.CompilerParams(use_tc_tiling_on_sc=False),
    )
    def k(idx_hbm, tab_hbm, out_hbm, idx_v, raw_v, rows_v, gsem, ssem):
        wid = lax.axis_index("s") * _NC + lax.axis_index("c")
        base = wid * (nch * _CH)
        pltpu.sync_copy(idx_hbm.at[wid], idx_v)

        def start_gather(j, b):
            pltpu.async_copy(tab_hbm.at[idx_v.at[j]], raw_v.at[b], gsem.at[b])

        def wait_gather(j, b):
            pltpu.make_async_copy(
                tab_hbm.at[idx_v.at[j]], raw_v.at[b], gsem.at[b]
            ).wait()

        def start_scatter(j, b):
            pltpu.async_copy(
                rows_v.at[b], out_hbm.at[pl.ds(base + j * _CH, _CH)], ssem.at[b]
            )

        def wait_scatter(j, b):
            pltpu.make_async_copy(
                rows_v.at[b], out_hbm.at[pl.ds(base + j * _CH, _CH)], ssem.at[b]
            ).wait()

        hi_mask = jnp.full((16,), 0xFFFF0000, dtype=jnp.uint32)
        shift16 = jnp.full((16,), 16, dtype=jnp.uint32)

        def convert_buf(b):
            src = raw_v.at[b]
            dst = rows_v.at[b]

            def body(r, carry):
                for g in range(_NGRP):
                    w = src[r, pl.ds(g * 16, 16)]
                    lo = lax.bitcast_convert_type(lax.shift_left(w, shift16), jnp.float32)
                    hi = lax.bitcast_convert_type(lax.bitwise_and(w, hi_mask), jnp.float32)
                    dst[r, pl.ds(g * 32, 16)] = lo * SCALE
                    dst[r, pl.ds(g * 32 + 16, 16)] = hi * SCALE
                return carry

            lax.fori_loop(0, _CH, body, 0)

        def full_step(m, b, retire_scatter, issue_gather):
            wait_gather(m, b)
            if retire_scatter:
                wait_scatter(m - _NBUF, b)
            convert_buf(b)
            if issue_gather:
                start_gather(m + _NBUF, b)
            start_scatter(m, b)

        for b in range(_NBUF):
            start_gather(b, b)

        for b in range(_NBUF):
            full_step(b, b, False, True)

        def group(g, carry):
            for b in range(_NBUF):
                full_step(g * _NBUF + b, b, True, True)
            return carry

        lax.fori_loop(1, nch // _NBUF - 1, group, 0)

        for b in range(_NBUF):
            full_step((nch - _NBUF) + b, b, True, False)
        for b in range(_NBUF):
            wait_scatter((nch - _NBUF) + b, b)

    return k(idx, packed)


def kernel(x, table):
    b_total = x.size
    assert b_total % (_NW * _CH) == 0
    nch = b_total // (_NW * _CH)
    assert nch % _NBUF == 0
    packed = _pack_table(table)
    idx = x.reshape(_NW, nch, _CH).astype(jnp.int32)
    out = _gather(idx, packed, nch)
    return out.reshape(x.shape + (D_EMB,))


# speedup vs baseline: 2.3717x; 2.3717x over previous
import functools
import math

import jax
import jax.numpy as jnp
from jax import lax
from jax.experimental import pallas as pl
from jax.experimental.pallas import tpu as pltpu
from jax.experimental.pallas import tpu_sc as plsc

D_EMB = 128
SCALE = math.sqrt(float(D_EMB))

_info = plsc.get_sparse_core_info()
_NC = _info.num_cores
_NS = _info.num_subcores
_NW = _NC * _NS

_CH = 128
_NBUF = 5
_K = 2


@functools.partial(jax.jit, static_argnames=("nch",))
def _gather(idx, table, nch):
    b_total = _NW * nch * _CH
    mesh = plsc.VectorSubcoreMesh(core_axis_name="c", subcore_axis_name="s")

    @functools.partial(
        pl.kernel,
        mesh=mesh,
        out_type=jax.ShapeDtypeStruct((b_total, D_EMB), jnp.float32),
        scratch_types=[
            pltpu.VMEM((nch, _CH), jnp.int32),
            pltpu.VMEM((_NBUF, _CH, D_EMB), jnp.float32),
            pltpu.SemaphoreType.DMA((_NBUF,)),
            pltpu.SemaphoreType.DMA((_NBUF,)),
        ],
    )
    def k(idx_hbm, table_hbm, out_hbm, idx_v, rows_v, gsem, ssem):
        wid = lax.axis_index("s") * _NC + lax.axis_index("c")
        base = wid * (nch * _CH)
        pltpu.sync_copy(idx_hbm.at[wid], idx_v)

        def start_gather(j, b):
            pltpu.async_copy(table_hbm.at[idx_v.at[j]], rows_v.at[b], gsem.at[b])

        def wait_gather(j, b):
            pltpu.make_async_copy(
                table_hbm.at[idx_v.at[j]], rows_v.at[b], gsem.at[b]
            ).wait()

        def start_scatter(j, b):
            pltpu.async_copy(
                rows_v.at[b], out_hbm.at[pl.ds(base + j * _CH, _CH)], ssem.at[b]
            )

        def wait_scatter(j, b):
            pltpu.make_async_copy(
                rows_v.at[b], out_hbm.at[pl.ds(base + j * _CH, _CH)], ssem.at[b]
            ).wait()

        def scale_buf(b):
            buf = rows_v.at[b]

            def body(r, carry):
                for u in range(2):
                    for c in range(8):
                        sl = (r * 2 + u, pl.ds(c * 16, 16))
                        buf[sl] = buf[sl] * SCALE
                return carry

            lax.fori_loop(0, _CH // 2, body, 0)

        def full_step(m, b, issue_gather):
            bp = (b - _K) % _NBUF
            wait_scatter(m - _K, bp)
            if issue_gather:
                start_gather(m - _K + _NBUF, bp)
            wait_gather(m, b)
            scale_buf(b)
            start_scatter(m, b)

        for b in range(_NBUF):
            start_gather(b, b)

        for b in range(_K):
            wait_gather(b, b)
            scale_buf(b)
            start_scatter(b, b)
        for b in range(_K, _NBUF):
            full_step(b, b, True)

        def group(g, carry):
            for b in range(_NBUF):
                full_step(g * _NBUF + b, b, True)
            return carry

        lax.fori_loop(1, nch // _NBUF - 1, group, 0)

        gl = nch // _NBUF - 1
        for b in range(_NBUF):
            m = gl * _NBUF + b
            full_step(m, b, m - _K + _NBUF < nch)
        for m in range(nch - _K, nch):
            wait_scatter(m, m % _NBUF)

    return k(idx, table)


def kernel(x, table):
    b_total = x.size
    assert b_total % (_NW * _CH) == 0
    nch = b_total // (_NW * _CH)
    assert nch % _NBUF == 0
    idx = x.reshape(_NW, nch, _CH).astype(jnp.int32)
    out = _gather(idx, table, nch)
    return out.reshape(x.shape + (D_EMB,))
